# Initial kernel scaffold; baseline (speedup 1.0000x reference)
import jax
import jax.numpy as jnp
from jax.experimental import pallas as pl


def kernel(token_ids, W):
    raise NotImplementedError("write your pallas kernel here")


# speedup vs baseline: 1.3082x; 1.3082x over previous
import functools

import jax
import jax.numpy as jnp
from jax import lax
from jax.experimental import pallas as pl
from jax.experimental.pallas import tpu as pltpu
from jax.experimental.pallas import tpu_sc as plsc

NUM_EMBEDDINGS = 1000000
D = 32
BATCH = 4096
HIST_LEN = 200
B = BATCH * HIST_LEN

NC = 2
NS = 16
NW = NC * NS
G = 128
GROUPS = B // (NW * G)


def _make_sc_lookup():
    mesh = plsc.VectorSubcoreMesh(core_axis_name="c", subcore_axis_name="s")

    @functools.partial(
        pl.kernel,
        mesh=mesh,
        out_type=jax.ShapeDtypeStruct((B, D), jnp.float32),
        compiler_params=pltpu.CompilerParams(use_tc_tiling_on_sc=False),
        scratch_types=[
            pltpu.VMEM((GROUPS, G), jnp.int32),
            pltpu.VMEM((G, D), jnp.float32),
            pltpu.SemaphoreType.DMA,
        ],
    )
    def lookup(idx_hbm, table_hbm, out_hbm, idx_v, rows_v, gsem):
        wid = lax.axis_index("s") * NC + lax.axis_index("c")
        pltpu.sync_copy(idx_hbm.at[pl.ds(wid * GROUPS, GROUPS)], idx_v)

        def step(j, carry):
            pltpu.async_copy(table_hbm.at[idx_v.at[j]], rows_v, gsem).wait()
            pltpu.sync_copy(rows_v, out_hbm.at[pl.ds((wid * GROUPS + j) * G, G)])
            return carry

        lax.fori_loop(0, GROUPS, step, 0)

    return lookup


_lookup = _make_sc_lookup()


def kernel(token_ids, W):
    idx = token_ids.reshape(NW * GROUPS, G).astype(jnp.int32)
    out = _lookup(idx, W)
    return out.reshape(BATCH, HIST_LEN, D)


# speedup vs baseline: 1.4932x; 1.1415x over previous
import functools

import jax
import jax.numpy as jnp
from jax import lax
from jax.experimental import pallas as pl
from jax.experimental.pallas import tpu as pltpu
from jax.experimental.pallas import tpu_sc as plsc

NUM_EMBEDDINGS = 1000000
D = 32
BATCH = 4096
HIST_LEN = 200
B = BATCH * HIST_LEN

NC = 2
NS = 16
NW = NC * NS
G = 128
GROUPS = B // (NW * G)
KG = 10
HALVES = GROUPS // KG


def _make_sc_lookup():
    mesh = plsc.VectorSubcoreMesh(core_axis_name="c", subcore_axis_name="s")

    @functools.partial(
        pl.kernel,
        mesh=mesh,
        out_type=jax.ShapeDtypeStruct((NW * HALVES, KG * G, D), jnp.float32),
        compiler_params=pltpu.CompilerParams(use_tc_tiling_on_sc=False),
        scratch_types=[
            pltpu.VMEM((GROUPS, G), jnp.int32),
            pltpu.VMEM((2, KG * G, D), jnp.float32),
            pltpu.SemaphoreType.DMA,
            pltpu.SemaphoreType.DMA,
        ],
    )
    def lookup(idx_hbm, table_hbm, out_hbm, idx_v, rows_v, gsem, osem):
        wid = lax.axis_index("s") * NC + lax.axis_index("c")
        pltpu.sync_copy(idx_hbm.at[pl.ds(wid * GROUPS, GROUPS)], idx_v)

        def half(h, carry):
            b = lax.rem(h, 2)

            @pl.when(h >= 2)
            def _():
                pltpu.make_async_copy(
                    rows_v.at[b], out_hbm.at[wid * HALVES + h - 2], osem
                ).wait()

            for jj in range(KG):
                pltpu.async_copy(
                    table_hbm.at[idx_v.at[h * KG + jj]],
                    rows_v.at[b, pl.ds(jj * G, G)],
                    gsem,
                )
            for jj in range(KG):
                pltpu.make_async_copy(
                    table_hbm.at[idx_v.at[h * KG + jj]],
                    rows_v.at[b, pl.ds(jj * G, G)],
                    gsem,
                ).wait()

            pltpu.async_copy(rows_v.at[b], out_hbm.at[wid * HALVES + h], osem)
            return carry

        lax.fori_loop(0, HALVES, half, 0)

        for h in (HALVES - 2, HALVES - 1):
            pltpu.make_async_copy(
                rows_v.at[h % 2], out_hbm.at[wid * HALVES + h], osem
            ).wait()

    return lookup


_lookup = _make_sc_lookup()


def kernel(token_ids, W):
    idx = token_ids.reshape(NW * GROUPS, G).astype(jnp.int32)
    out = _lookup(idx, W)
    return out.reshape(BATCH, HIST_LEN, D)


# speedup vs baseline: 1.5700x; 1.0515x over previous
import functools

import jax
import jax.numpy as jnp
from jax import lax
from jax.experimental import pallas as pl
from jax.experimental.pallas import tpu as pltpu
from jax.experimental.pallas import tpu_sc as plsc

NUM_EMBEDDINGS = 1000000
D = 32
BATCH = 4096
HIST_LEN = 200
B = BATCH * HIST_LEN

NC = 2
NS = 16
NW = NC * NS
G = 128
GROUPS = B // (NW * G)
KG = 10
HALVES = GROUPS // KG


def _make_sc_lookup():
    mesh = plsc.VectorSubcoreMesh(core_axis_name="c", subcore_axis_name="s")

    @functools.partial(
        pl.kernel,
        mesh=mesh,
        out_type=jax.ShapeDtypeStruct((NW * HALVES, KG * G, D), jnp.float32),
        compiler_params=pltpu.CompilerParams(use_tc_tiling_on_sc=False),
        scratch_types=[
            pltpu.VMEM((GROUPS, G), jnp.int32),
            pltpu.VMEM((2, KG * G, D), jnp.float32),
            pltpu.SemaphoreType.DMA,
            pltpu.SemaphoreType.DMA,
        ],
    )
    def lookup(idx_hbm, table_hbm, out_hbm, idx_v, rows_v, gsem, osem):
        wid = lax.axis_index("s") * NC + lax.axis_index("c")
        pltpu.sync_copy(idx_hbm.at[pl.ds(wid * GROUPS, GROUPS)], idx_v)

        def half(h, carry):
            b = lax.rem(h, 2)

            @pl.when(h >= 2)
            def _():
                pltpu.make_async_copy(
                    rows_v.at[b], out_hbm.at[wid * HALVES + h - 2], osem
                ).wait()

            for jj in range(KG):
                pltpu.async_copy(
                    table_hbm.at[idx_v.at[h * KG + jj]],
                    rows_v.at[b, pl.ds(jj * G, G)],
                    gsem,
                )
            for jj in range(KG):
                pltpu.make_async_copy(
                    table_hbm.at[idx_v.at[h * KG + jj]],
                    rows_v.at[b, pl.ds(jj * G, G)],
                    gsem,
                ).wait()

            pltpu.async_copy(rows_v.at[b], out_hbm.at[wid * HALVES + h], osem)
            return carry

        lax.fori_loop(0, HALVES, half, 0)

        for h in (HALVES - 2, HALVES - 1):
            pltpu.make_async_copy(
                rows_v.at[h % 2], out_hbm.at[wid * HALVES + h], osem
            ).wait()

    return lookup


_lookup = _make_sc_lookup()


def kernel(token_ids, W):
    idx = jnp.transpose(token_ids).reshape(NW * GROUPS, G).astype(jnp.int32)
    out = _lookup(idx, W)
    return jnp.transpose(out.reshape(HIST_LEN, BATCH, D), (1, 0, 2))
